# Initial kernel scaffold; baseline (speedup 1.0000x reference)
import jax
import jax.numpy as jnp
from jax.experimental import pallas as pl


def kernel(inputs, embedding):
    raise NotImplementedError("write your pallas kernel here")


# speedup vs baseline: 1.5886x; 1.5886x over previous
import jax
import jax.numpy as jnp
from jax.experimental import pallas as pl

_BLK = 1024


def _vq_body(x_ref, e_ref, q_ref, idx_ref, sq_ref):
    x = x_ref[...]
    e = e_ref[...]
    k = e.shape[1]
    xe = jax.lax.dot_general(
        x, e, (((1,), (0,)), ((), ())), preferred_element_type=jnp.float32
    )
    x2 = jnp.sum(x * x, axis=1, keepdims=True)
    e2 = jnp.sum(e * e, axis=0, keepdims=True)
    distance = x2 - 2.0 * xe + e2
    idx = jnp.argmax(-distance, axis=1)
    onehot = (
        jax.lax.broadcasted_iota(jnp.int32, (x.shape[0], k), 1) == idx[:, None]
    ).astype(jnp.float32)
    quant = jax.lax.dot_general(
        onehot, e, (((1,), (1,)), ((), ())), preferred_element_type=jnp.float32
    )
    q_ref[...] = x + (quant - x)
    idx_ref[...] = idx.reshape(1, 1, -1)

    @pl.when(pl.program_id(0) == 0)
    def _init():
        sq_ref[...] = jnp.zeros_like(sq_ref)

    sq_ref[...] += jnp.sum((quant - x) ** 2).reshape(1, 1)


def kernel(inputs, embedding):
    d = embedding.shape[0]
    k = embedding.shape[1]
    tokens = inputs.size // d
    x_flat = inputs.reshape(tokens, d)
    nb = tokens // _BLK

    q, idx, sq = pl.pallas_call(
        _vq_body,
        grid=(nb,),
        in_specs=[
            pl.BlockSpec((_BLK, d), lambda i: (i, 0)),
            pl.BlockSpec((d, k), lambda i: (0, 0)),
        ],
        out_specs=[
            pl.BlockSpec((_BLK, d), lambda i: (i, 0)),
            pl.BlockSpec((1, 1, _BLK), lambda i: (i, 0, 0)),
            pl.BlockSpec((1, 1), lambda i: (0, 0)),
        ],
        out_shape=[
            jax.ShapeDtypeStruct((tokens, d), jnp.float32),
            jax.ShapeDtypeStruct((nb, 1, _BLK), jnp.int32),
            jax.ShapeDtypeStruct((1, 1), jnp.float32),
        ],
    )(x_flat, embedding)

    quantize = q.reshape(inputs.shape)
    diff = (sq[0, 0] / jnp.float32(inputs.size)).reshape(())
    indices = idx.reshape(inputs.shape[:-1])
    return (quantize, diff, indices)


# speedup vs baseline: 1.6993x; 1.0697x over previous
import jax
import jax.numpy as jnp
from jax.experimental import pallas as pl

_BLK = 1024


def _vq_body(x_ref, e_ref, q_ref, idx_ref, sq_ref):
    x = x_ref[...]
    e = e_ref[...]
    k = e.shape[1]
    xe2 = jax.lax.dot_general(
        x + x, e, (((1,), (0,)), ((), ())), preferred_element_type=jnp.float32
    )
    e2 = jnp.sum(e * e, axis=0, keepdims=True)
    s = xe2 - e2
    idx = jnp.argmax(s, axis=1)
    onehot = (
        jax.lax.broadcasted_iota(jnp.int32, (x.shape[0], k), 1) == idx[:, None]
    ).astype(jnp.float32)
    quant = jax.lax.dot_general(
        onehot, e, (((1,), (1,)), ((), ())), preferred_element_type=jnp.float32
    )
    q_ref[...] = x + (quant - x)
    idx_ref[...] = idx.reshape(1, 1, -1)

    @pl.when(pl.program_id(0) == 0)
    def _init():
        sq_ref[...] = jnp.zeros_like(sq_ref)

    sq_ref[...] += jnp.sum((quant - x) ** 2).reshape(1, 1)


def kernel(inputs, embedding):
    d = embedding.shape[0]
    k = embedding.shape[1]
    tokens = inputs.size // d
    x_flat = inputs.reshape(tokens, d)
    nb = tokens // _BLK

    q, idx, sq = pl.pallas_call(
        _vq_body,
        grid=(nb,),
        in_specs=[
            pl.BlockSpec((_BLK, d), lambda i: (i, 0)),
            pl.BlockSpec((d, k), lambda i: (0, 0)),
        ],
        out_specs=[
            pl.BlockSpec((_BLK, d), lambda i: (i, 0)),
            pl.BlockSpec((1, 1, _BLK), lambda i: (i, 0, 0)),
            pl.BlockSpec((1, 1), lambda i: (0, 0)),
        ],
        out_shape=[
            jax.ShapeDtypeStruct((tokens, d), jnp.float32),
            jax.ShapeDtypeStruct((nb, 1, _BLK), jnp.int32),
            jax.ShapeDtypeStruct((1, 1), jnp.float32),
        ],
    )(x_flat, embedding)

    quantize = q.reshape(inputs.shape)
    diff = (sq[0, 0] / jnp.float32(inputs.size)).reshape(())
    indices = idx.reshape(inputs.shape[:-1])
    return (quantize, diff, indices)


# speedup vs baseline: 1.8777x; 1.1050x over previous
import jax
import jax.numpy as jnp
from jax.experimental import pallas as pl

_BLK = 2048


def _vq_body(x_ref, e_ref, q_ref, idx_ref, sq_ref):
    x = x_ref[...]
    e = e_ref[...]
    k = e.shape[1]
    xe2 = jax.lax.dot_general(
        x + x, e, (((1,), (0,)), ((), ())), preferred_element_type=jnp.float32
    )
    e2 = jnp.sum(e * e, axis=0, keepdims=True)
    s = xe2 - e2
    idx = jnp.argmax(s, axis=1)
    onehot = (
        jax.lax.broadcasted_iota(jnp.int32, (x.shape[0], k), 1) == idx[:, None]
    ).astype(jnp.float32)
    quant = jax.lax.dot_general(
        onehot, e, (((1,), (1,)), ((), ())), preferred_element_type=jnp.float32
    )
    q_ref[...] = x + (quant - x)
    idx_ref[...] = idx.reshape(1, 1, -1)

    @pl.when(pl.program_id(0) == 0)
    def _init():
        sq_ref[...] = jnp.zeros_like(sq_ref)

    sq_ref[...] += jnp.sum((quant - x) ** 2).reshape(1, 1)


def kernel(inputs, embedding):
    d = embedding.shape[0]
    k = embedding.shape[1]
    tokens = inputs.size // d
    x_flat = inputs.reshape(tokens, d)
    nb = tokens // _BLK

    q, idx, sq = pl.pallas_call(
        _vq_body,
        grid=(nb,),
        in_specs=[
            pl.BlockSpec((_BLK, d), lambda i: (i, 0)),
            pl.BlockSpec((d, k), lambda i: (0, 0)),
        ],
        out_specs=[
            pl.BlockSpec((_BLK, d), lambda i: (i, 0)),
            pl.BlockSpec((1, 1, _BLK), lambda i: (i, 0, 0)),
            pl.BlockSpec((1, 1), lambda i: (0, 0)),
        ],
        out_shape=[
            jax.ShapeDtypeStruct((tokens, d), jnp.float32),
            jax.ShapeDtypeStruct((nb, 1, _BLK), jnp.int32),
            jax.ShapeDtypeStruct((1, 1), jnp.float32),
        ],
    )(x_flat, embedding)

    quantize = q.reshape(inputs.shape)
    diff = (sq[0, 0] / jnp.float32(inputs.size)).reshape(())
    indices = idx.reshape(inputs.shape[:-1])
    return (quantize, diff, indices)


# speedup vs baseline: 1.9257x; 1.0256x over previous
import jax
import jax.numpy as jnp
from jax.experimental import pallas as pl

_BLK = 4096


def _vq_body(x_ref, e_ref, q_ref, idx_ref, sq_ref):
    x = x_ref[...]
    e = e_ref[...]
    k = e.shape[1]
    xe2 = jax.lax.dot_general(
        x + x, e, (((1,), (0,)), ((), ())), preferred_element_type=jnp.float32
    )
    e2 = jnp.sum(e * e, axis=0, keepdims=True)
    s = xe2 - e2
    idx = jnp.argmax(s, axis=1)
    onehot = (
        jax.lax.broadcasted_iota(jnp.int32, (x.shape[0], k), 1) == idx[:, None]
    ).astype(jnp.float32)
    quant = jax.lax.dot_general(
        onehot, e, (((1,), (1,)), ((), ())), preferred_element_type=jnp.float32
    )
    q_ref[...] = x + (quant - x)
    idx_ref[...] = idx.reshape(1, 1, -1)

    @pl.when(pl.program_id(0) == 0)
    def _init():
        sq_ref[...] = jnp.zeros_like(sq_ref)

    sq_ref[...] += jnp.sum((quant - x) ** 2).reshape(1, 1)


def kernel(inputs, embedding):
    d = embedding.shape[0]
    k = embedding.shape[1]
    tokens = inputs.size // d
    x_flat = inputs.reshape(tokens, d)
    nb = tokens // _BLK

    q, idx, sq = pl.pallas_call(
        _vq_body,
        grid=(nb,),
        in_specs=[
            pl.BlockSpec((_BLK, d), lambda i: (i, 0)),
            pl.BlockSpec((d, k), lambda i: (0, 0)),
        ],
        out_specs=[
            pl.BlockSpec((_BLK, d), lambda i: (i, 0)),
            pl.BlockSpec((1, 1, _BLK), lambda i: (i, 0, 0)),
            pl.BlockSpec((1, 1), lambda i: (0, 0)),
        ],
        out_shape=[
            jax.ShapeDtypeStruct((tokens, d), jnp.float32),
            jax.ShapeDtypeStruct((nb, 1, _BLK), jnp.int32),
            jax.ShapeDtypeStruct((1, 1), jnp.float32),
        ],
    )(x_flat, embedding)

    quantize = q.reshape(inputs.shape)
    diff = (sq[0, 0] / jnp.float32(inputs.size)).reshape(())
    indices = idx.reshape(inputs.shape[:-1])
    return (quantize, diff, indices)


# speedup vs baseline: 2.0551x; 1.0672x over previous
import jax
import jax.numpy as jnp
from jax.experimental import pallas as pl

_RB = 8
_CB = 512


def _vq_body(x_ref, e_ref, q_ref, idx_ref, sq_ref):
    x = x_ref[...].reshape(_RB * _CB, x_ref.shape[-1])
    e = e_ref[...]
    k = e.shape[1]
    xe2 = jax.lax.dot_general(
        x + x, e, (((1,), (0,)), ((), ())), preferred_element_type=jnp.float32
    )
    e2 = jnp.sum(e * e, axis=0, keepdims=True)
    s = xe2 - e2
    idx = jnp.argmax(s, axis=1)
    onehot = (
        jax.lax.broadcasted_iota(jnp.int32, (x.shape[0], k), 1) == idx[:, None]
    ).astype(jnp.float32)
    quant = jax.lax.dot_general(
        onehot, e, (((1,), (1,)), ((), ())), preferred_element_type=jnp.float32
    )
    q_ref[...] = (x + (quant - x)).reshape(q_ref.shape)
    idx_ref[...] = idx.reshape(idx_ref.shape)

    i, j = pl.program_id(0), pl.program_id(1)
    ni, nj = pl.num_programs(0), pl.num_programs(1)

    @pl.when((i == 0) & (j == 0))
    def _init():
        sq_ref[...] = jnp.zeros_like(sq_ref)

    sq_ref[...] += jnp.sum((quant - x) ** 2).reshape(1, 1)

    @pl.when((i == ni - 1) & (j == nj - 1))
    def _fin():
        sq_ref[...] = sq_ref[...] / jnp.float32(ni * nj * x.size)


def kernel(inputs, embedding):
    d = embedding.shape[0]
    k = embedding.shape[1]
    rows, cols = inputs.shape[0], inputs.shape[1]

    quantize, idx, sq = pl.pallas_call(
        _vq_body,
        grid=(rows // _RB, cols // _CB),
        in_specs=[
            pl.BlockSpec((_RB, _CB, d), lambda i, j: (i, j, 0)),
            pl.BlockSpec((d, k), lambda i, j: (0, 0)),
        ],
        out_specs=[
            pl.BlockSpec((_RB, _CB, d), lambda i, j: (i, j, 0)),
            pl.BlockSpec((_RB, _CB), lambda i, j: (i, j)),
            pl.BlockSpec((1, 1), lambda i, j: (0, 0)),
        ],
        out_shape=[
            jax.ShapeDtypeStruct((rows, cols, d), jnp.float32),
            jax.ShapeDtypeStruct((rows, cols), jnp.int32),
            jax.ShapeDtypeStruct((1, 1), jnp.float32),
        ],
    )(inputs, embedding)

    return (quantize, sq.reshape(()), idx)


# speedup vs baseline: 2.2238x; 1.0821x over previous
import jax
import jax.numpy as jnp
from jax.experimental import pallas as pl

_RB = 8
_CB = 512


def _vq_body(x_ref, e_ref, q_ref, idx_ref, sq_ref):
    x = x_ref[...].reshape(_RB * _CB, x_ref.shape[-1])
    e = e_ref[...]
    d, k = e.shape
    b = x.shape[0]
    xe2 = jax.lax.dot_general(
        x + x, e, (((1,), (0,)), ((), ())), preferred_element_type=jnp.float32
    )
    e2 = jnp.sum(e * e, axis=0, keepdims=True)
    s = xe2 - e2
    m = jnp.max(s, axis=1, keepdims=True)
    maskf = (s == m).astype(jnp.float32)
    iota_row = jax.lax.broadcasted_iota(jnp.int32, (1, k), 1).astype(jnp.float32)
    ones_row = jnp.ones((1, k), jnp.float32)
    aug = jnp.concatenate([e, iota_row, ones_row], axis=0)
    r = jax.lax.dot_general(
        maskf, aug, (((1,), (1,)), ((), ())), preferred_element_type=jnp.float32
    )
    quant = r[:, :d]
    idx = r[:, d : d + 1].astype(jnp.int32)
    cnt = r[:, d + 1 : d + 2]
    tie = jnp.any(cnt != 1.0)

    i, j = pl.program_id(0), pl.program_id(1)
    ni, nj = pl.num_programs(0), pl.num_programs(1)

    @pl.when((i == 0) & (j == 0))
    def _init():
        sq_ref[...] = jnp.zeros_like(sq_ref)

    @pl.when(jnp.logical_not(tie))
    def _fast():
        q_ref[...] = (x + (quant - x)).reshape(q_ref.shape)
        idx_ref[...] = idx.reshape(idx_ref.shape)
        sq_ref[...] += jnp.sum((quant - x) ** 2).reshape(1, 1)

    @pl.when(tie)
    def _slow():
        idx2 = jnp.argmax(s, axis=1)
        onehot = (
            jax.lax.broadcasted_iota(jnp.int32, (b, k), 1) == idx2[:, None]
        ).astype(jnp.float32)
        quant2 = jax.lax.dot_general(
            onehot, e, (((1,), (1,)), ((), ())), preferred_element_type=jnp.float32
        )
        q_ref[...] = (x + (quant2 - x)).reshape(q_ref.shape)
        idx_ref[...] = idx2.reshape(idx_ref.shape)
        sq_ref[...] += jnp.sum((quant2 - x) ** 2).reshape(1, 1)

    @pl.when((i == ni - 1) & (j == nj - 1))
    def _fin():
        sq_ref[...] = sq_ref[...] / jnp.float32(ni * nj * b * d)


def kernel(inputs, embedding):
    d = embedding.shape[0]
    k = embedding.shape[1]
    rows, cols = inputs.shape[0], inputs.shape[1]

    quantize, idx, sq = pl.pallas_call(
        _vq_body,
        grid=(rows // _RB, cols // _CB),
        in_specs=[
            pl.BlockSpec((_RB, _CB, d), lambda i, j: (i, j, 0)),
            pl.BlockSpec((d, k), lambda i, j: (0, 0)),
        ],
        out_specs=[
            pl.BlockSpec((_RB, _CB, d), lambda i, j: (i, j, 0)),
            pl.BlockSpec((_RB, _CB), lambda i, j: (i, j)),
            pl.BlockSpec((1, 1), lambda i, j: (0, 0)),
        ],
        out_shape=[
            jax.ShapeDtypeStruct((rows, cols, d), jnp.float32),
            jax.ShapeDtypeStruct((rows, cols), jnp.int32),
            jax.ShapeDtypeStruct((1, 1), jnp.float32),
        ],
    )(inputs, embedding)

    return (quantize, sq.reshape(()), idx)
